# baseline (speedup 1.0000x reference)
import jax
import jax.numpy as jnp
from jax import lax
from jax.experimental import pallas as pl
from jax.experimental.pallas import tpu as pltpu
from jax.experimental.pallas import tpu_sc as plsc

N, E, C = 10000, 320000, 128
D = 136
NC, NS, NW = 2, 16, 32
K = 80
EPW = E // NW
STEPS = EPW // K
NP = 10240
RPS = NP // NS
ROWB = 1000
LANES = 16


def _stage1_body(batch_ref, emb_ref, w_ref, ai_ref, aj_ref, aei_ref, aej_ref,
                 x_ref, xbf_ref, aij_ref, bmax_ref):
    i = pl.program_id(0)
    x = lax.dot_general(batch_ref[...], w_ref[...], (((1,), (1,)), ((), ())),
                        preferred_element_type=jnp.float32)
    x_ref[...] = x
    parts = [
        jnp.stack((x[:, o:o + LANES], x[:, o + LANES:o + 2 * LANES]),
                  axis=-1).reshape(ROWB, 2 * LANES)
        for o in range(0, C, 2 * LANES)
    ]
    xbf_ref[...] = jnp.concatenate(parts, axis=1).astype(jnp.bfloat16)
    emb = emb_ref[...]
    ai = jnp.sum(x * ai_ref[...][None, :], axis=1) + \
        jnp.sum(emb * aei_ref[...][None, :], axis=1)
    aj = jnp.sum(x * aj_ref[...][None, :], axis=1) + \
        jnp.sum(emb * aej_ref[...][None, :], axis=1)
    aij_ref[...] = jnp.stack([ai, aj], axis=1)

    @pl.when(i == 0)
    def _():
        bmax_ref[...] = jnp.full((1, 2), -jnp.inf, jnp.float32)

    m = jnp.stack([jnp.max(ai), jnp.max(aj)])[None, :]
    bmax_ref[...] = jnp.maximum(bmax_ref[...], m)


_stage1 = pl.pallas_call(
    _stage1_body,
    grid=(N // ROWB,),
    in_specs=[
        pl.BlockSpec((ROWB, C), lambda i: (i, 0)),
        pl.BlockSpec((ROWB, C), lambda i: (i, 0)),
        pl.BlockSpec((C, C), lambda i: (0, 0)),
        pl.BlockSpec((C,), lambda i: (0,)),
        pl.BlockSpec((C,), lambda i: (0,)),
        pl.BlockSpec((C,), lambda i: (0,)),
        pl.BlockSpec((C,), lambda i: (0,)),
    ],
    out_specs=[
        pl.BlockSpec((ROWB, C), lambda i: (i, 0)),
        pl.BlockSpec((ROWB, C), lambda i: (i, 0)),
        pl.BlockSpec((ROWB, 2), lambda i: (i, 0)),
        pl.BlockSpec((1, 2), lambda i: (0, 0)),
    ],
    out_shape=[
        jax.ShapeDtypeStruct((N, C), jnp.float32),
        jax.ShapeDtypeStruct((N, C), jnp.bfloat16),
        jax.ShapeDtypeStruct((N, 2), jnp.float32),
        jax.ShapeDtypeStruct((1, 2), jnp.float32),
    ],
)


def _lane_bcast(vec, lane):
    return lax.gather(
        vec, jnp.full((LANES, 1), lane, jnp.int32),
        lax.GatherDimensionNumbers(offset_dims=(), collapsed_slice_dims=(0,),
                                   start_index_map=(0,)),
        (1,), mode=lax.GatherScatterMode.PROMISE_IN_BOUNDS)


def _lane_shuf(vec, idx):
    return lax.gather(
        vec, idx[:, None],
        lax.GatherDimensionNumbers(offset_dims=(), collapsed_slice_dims=(0,),
                                   start_index_map=(0,)),
        (1,), mode=lax.GatherScatterMode.PROMISE_IN_BOUNDS)


def _sc_body(xbf_hbm, aij_hbm, src_hbm, dst_hbm, bvec_hbm, acc_hbm,
             aij_t, sidx0, didx0, sdidx0, gbf0, sidx1, didx1, sdidx1, gbf1,
             srow, w_t, b_t, acc_sh, sg0, sg1, si0, si1, ss):
    cid = lax.axis_index("c")
    sid = lax.axis_index("s")
    wid = sid * NC + cid
    bufs = ((sidx0, didx0, sdidx0, gbf0, sg0, si0),
            (sidx1, didx1, sdidx1, gbf1, sg1, si1))

    pltpu.sync_copy(aij_hbm, aij_t)
    pltpu.sync_copy(bvec_hbm, b_t)

    def _zrow(r, carry):
        for c in range(C // LANES):
            srow[r, pl.ds(c * LANES, LANES)] = \
                jnp.zeros((LANES,), jnp.float32)
        srow[r, pl.ds(D - LANES, LANES)] = jnp.zeros((LANES,), jnp.float32)
        return carry

    lax.fori_loop(0, K, _zrow, 0)
    for t in range(RPS // K):
        pltpu.sync_copy(srow, acc_sh.at[pl.ds(sid * RPS + t * K, K)])
    plsc.subcore_barrier()

    o16 = jnp.ones((LANES,), jnp.int32)
    iota = lax.iota(jnp.int32, LANES)
    rot8 = (iota + 8) & 15
    mhi = jnp.full((LANES,), -65536, jnp.int32)
    ebase = wid * EPW

    def _weights(sidx, didx):
        bv = b_t[...]
        for v in range(K // LANES):
            s16 = sidx[pl.ds(v * LANES, LANES)]
            d16 = didx[pl.ds(v * LANES, LANES)]
            ai = plsc.load_gather(aij_t, (d16 * 2,))
            aj = plsc.load_gather(aij_t, (s16 * 2 + o16,))
            s = ai + aj
            al = jnp.where(s >= 0, s, 0.2 * s)
            w = jnp.exp(al - bv)
            w = jnp.where(s16 == d16, jnp.float32(0.0), w)
            w_t[pl.ds(v * LANES, LANES)] = w

    def _scale_rows(gbf):
        def _scale(v, c2):
            w16 = w_t[pl.ds(v * LANES, LANES)]
            for lane in range(LANES):
                wl = _lane_bcast(w16, lane)
                j = v * LANES + lane
                hi_last = None
                for g in range(C // (2 * LANES)):
                    packed = gbf[j, pl.ds(g * 2 * LANES, 2 * LANES)]
                    word = plsc.bitcast(packed, jnp.int32)
                    lo = plsc.bitcast(word << 16, jnp.float32) * wl
                    hi = plsc.bitcast(word & mhi, jnp.float32) * wl
                    srow[j, pl.ds(g * 2 * LANES, LANES)] = lo
                    srow[j, pl.ds(g * 2 * LANES + LANES, LANES)] = hi
                    hi_last = hi
                sh = _lane_shuf(hi_last, rot8)
                tail = jnp.where(iota < 8, sh,
                                 jnp.where(iota == 8, wl,
                                           jnp.zeros((LANES,), jnp.float32)))
                srow[j, pl.ds(D - LANES, LANES)] = tail
            return c2

        lax.fori_loop(0, K // LANES, _scale, 0)

    sidxP, didxP, _, gbfP, sgP, _ = bufs[0]
    pltpu.sync_copy(src_hbm.at[pl.ds(ebase, K)], sidxP)
    pltpu.sync_copy(dst_hbm.at[pl.ds(ebase, K)], didxP)
    pltpu.async_copy(xbf_hbm.at[sidxP], gbfP, sgP)

    def _iter(i2, b):
        sidx, didx, sdidx, gbf, sg, si = bufs[b]
        osidx, odidx, _, ogbf, osg, osi = bufs[1 - b]
        nbase = ebase + (i2 + 1) * K

        pltpu.async_copy(src_hbm.at[pl.ds(nbase, K)], osidx, osi)
        pltpu.async_copy(dst_hbm.at[pl.ds(nbase, K)], odidx, osi)
        _weights(sidx, didx)
        for v in range(K // LANES):
            sdidx[pl.ds(v * LANES, LANES)] = didx[pl.ds(v * LANES, LANES)]
        pltpu.make_async_copy(xbf_hbm.at[sidx], gbf, sg).wait()
        if b == 0:
            @pl.when(i2 > 0)
            def _():
                pltpu.make_async_copy(srow, acc_sh.at[sdidx1], ss).wait()
        else:
            pltpu.make_async_copy(srow, acc_sh.at[sdidx0], ss).wait()
        _scale_rows(gbf)
        pltpu.make_async_copy(src_hbm.at[pl.ds(nbase, K)], osidx, osi).wait()
        pltpu.make_async_copy(dst_hbm.at[pl.ds(nbase, K)], odidx, osi).wait()
        pltpu.async_copy(xbf_hbm.at[osidx], ogbf, osg)
        pltpu.async_copy(srow, acc_sh.at[sdidx], ss, add=True)

    def _outer(g, carry):
        for b in range(2):
            _iter(g * 2 + b, b)
        return carry

    lax.fori_loop(0, (STEPS - 1) // 2, _outer, 0)

    sidxE, didxE, sdidxE, gbfE, sgE, _ = bufs[0]
    _weights(sidxE, didxE)
    for v in range(K // LANES):
        sdidxE[pl.ds(v * LANES, LANES)] = didxE[pl.ds(v * LANES, LANES)]
    pltpu.make_async_copy(xbf_hbm.at[sidxE], gbfE, sgE).wait()
    pltpu.make_async_copy(srow, acc_sh.at[sdidx1], ss).wait()
    _scale_rows(gbfE)
    pltpu.sync_copy(srow, acc_sh.at[sdidxE], add=True)

    plsc.subcore_barrier()
    pltpu.sync_copy(acc_sh.at[pl.ds(sid * RPS, RPS)],
                    acc_hbm.at[cid, pl.ds(sid * RPS, RPS)])


_sc_edges = pl.kernel(
    _sc_body,
    out_type=jax.ShapeDtypeStruct((NC, NP, D), jnp.float32),
    mesh=plsc.VectorSubcoreMesh(core_axis_name="c", subcore_axis_name="s"),
    compiler_params=pltpu.CompilerParams(needs_layout_passes=False, use_tc_tiling_on_sc=False),
    scratch_types=[
        pltpu.VMEM((2 * N,), jnp.float32),
        pltpu.VMEM((K,), jnp.int32),
        pltpu.VMEM((K,), jnp.int32),
        pltpu.VMEM((K,), jnp.int32),
        pltpu.VMEM((K, C), jnp.bfloat16),
        pltpu.VMEM((K,), jnp.int32),
        pltpu.VMEM((K,), jnp.int32),
        pltpu.VMEM((K,), jnp.int32),
        pltpu.VMEM((K, C), jnp.bfloat16),
        pltpu.VMEM((K, D), jnp.float32),
        pltpu.VMEM((K,), jnp.float32),
        pltpu.VMEM((LANES,), jnp.float32),
        pltpu.VMEM_SHARED((NP, D), jnp.float32),
        pltpu.SemaphoreType.DMA,
        pltpu.SemaphoreType.DMA,
        pltpu.SemaphoreType.DMA,
        pltpu.SemaphoreType.DMA,
        pltpu.SemaphoreType.DMA,
    ],
)


def _stage3_body(acc_ref, x_ref, aij_ref, bs_ref, bias_ref, gamma_ref,
                 beta_ref, out_ref):
    B = bs_ref[0, 0]
    x = x_ref[...]
    s = aij_ref[:, 0:1] + aij_ref[:, 1:2]
    al = jnp.where(s >= 0, s, 0.2 * s)
    exs = jnp.exp(al - B)
    num = acc_ref[0, :N, :C] + acc_ref[1, :N, :C] + exs * x
    den = acc_ref[0, :N, C:C + 1] + acc_ref[1, :N, C:C + 1] + exs
    o = num / jnp.maximum(den, 1e-16) + bias_ref[...][None, :]
    mean = jnp.mean(o, axis=0, keepdims=True)
    var = jnp.mean((o - mean) ** 2, axis=0, keepdims=True)
    o = (o - mean) / jnp.sqrt(var + 1e-5) * gamma_ref[...][None, :] + \
        beta_ref[...][None, :]
    out_ref[...] = jnp.maximum(o, 0.0)


_stage3 = pl.pallas_call(
    _stage3_body,
    out_shape=jax.ShapeDtypeStruct((N, C), jnp.float32),
)


def kernel(batch_mat, topk_edge, embedding, W, att_i, att_j, att_em_i,
           att_em_j, bias, gamma, beta):
    x, xbf, aij, bmax = _stage1(batch_mat, embedding, W, att_i, att_j,
                                att_em_i, att_em_j)
    ssum = bmax[0, 0] + bmax[0, 1]
    B = jnp.where(ssum >= 0, ssum, 0.2 * ssum)
    acc = _sc_edges(xbf, aij.reshape(2 * N), topk_edge[0], topk_edge[1],
                    jnp.broadcast_to(B, (LANES,)))
    return _stage3(acc, x, aij, B.reshape(1, 1), bias, gamma, beta)


# speedup vs baseline: 2.1785x; 2.1785x over previous
import jax
import jax.numpy as jnp
from jax import lax
from jax.experimental import pallas as pl
from jax.experimental.pallas import tpu as pltpu
from jax.experimental.pallas import tpu_sc as plsc

N, E, C = 10000, 320000, 128
D = 136
NC, NS, NW = 2, 16, 32
K = 80
EPW = E // NW
STEPS = EPW // K
NP = 10240
RPS = NP // NS
ROWB = 1000
LANES = 16


def _stage1_body(batch_ref, emb_ref, w_ref, ai_ref, aj_ref, aei_ref, aej_ref,
                 xaug_ref, aij_ref, bmax_ref):
    i = pl.program_id(0)
    x = lax.dot_general(batch_ref[...], w_ref[...], (((1,), (1,)), ((), ())),
                        preferred_element_type=jnp.float32)
    xaug_ref[:, :C] = x
    xaug_ref[:, C:C + 1] = jnp.ones((ROWB, 1), jnp.float32)
    xaug_ref[:, C + 1:] = jnp.zeros((ROWB, D - C - 1), jnp.float32)
    emb = emb_ref[...]
    ai = jnp.sum(x * ai_ref[...][None, :], axis=1) + \
        jnp.sum(emb * aei_ref[...][None, :], axis=1)
    aj = jnp.sum(x * aj_ref[...][None, :], axis=1) + \
        jnp.sum(emb * aej_ref[...][None, :], axis=1)
    aij_ref[...] = jnp.stack([ai, aj], axis=1)

    @pl.when(i == 0)
    def _():
        bmax_ref[...] = jnp.full((1, 2), -jnp.inf, jnp.float32)

    m = jnp.stack([jnp.max(ai), jnp.max(aj)])[None, :]
    bmax_ref[...] = jnp.maximum(bmax_ref[...], m)


_stage1 = pl.pallas_call(
    _stage1_body,
    grid=(N // ROWB,),
    in_specs=[
        pl.BlockSpec((ROWB, C), lambda i: (i, 0)),
        pl.BlockSpec((ROWB, C), lambda i: (i, 0)),
        pl.BlockSpec((C, C), lambda i: (0, 0)),
        pl.BlockSpec((C,), lambda i: (0,)),
        pl.BlockSpec((C,), lambda i: (0,)),
        pl.BlockSpec((C,), lambda i: (0,)),
        pl.BlockSpec((C,), lambda i: (0,)),
    ],
    out_specs=[
        pl.BlockSpec((ROWB, D), lambda i: (i, 0)),
        pl.BlockSpec((ROWB, 2), lambda i: (i, 0)),
        pl.BlockSpec((1, 2), lambda i: (0, 0)),
    ],
    out_shape=[
        jax.ShapeDtypeStruct((N, D), jnp.float32),
        jax.ShapeDtypeStruct((N, 2), jnp.float32),
        jax.ShapeDtypeStruct((1, 2), jnp.float32),
    ],
)


def _lane_bcast(vec, lane):
    return lax.gather(
        vec, jnp.full((LANES, 1), lane, jnp.int32),
        lax.GatherDimensionNumbers(offset_dims=(), collapsed_slice_dims=(0,),
                                   start_index_map=(0,)),
        (1,), mode=lax.GatherScatterMode.PROMISE_IN_BOUNDS)


def _sc_body(xaug_hbm, aij_hbm, edges_hbm, bvec_hbm, acc_hbm,
             aij_t, idx20, rows0, idx21, rows1, w_t, b_t,
             acc_sh, sg0, sg1, ss0, ss1, si0, si1):
    cid = lax.axis_index("c")
    sid = lax.axis_index("s")
    wid = sid * NC + cid
    bufs = ((idx20, rows0, sg0, ss0, si0),
            (idx21, rows1, sg1, ss1, si1))

    pltpu.sync_copy(aij_hbm, aij_t)
    pltpu.sync_copy(bvec_hbm, b_t)

    def _zrow(r, carry):
        for c in range(C // LANES):
            rows0[r, pl.ds(c * LANES, LANES)] = \
                jnp.zeros((LANES,), jnp.float32)
        rows0[r, pl.ds(D - LANES, LANES)] = jnp.zeros((LANES,), jnp.float32)
        return carry

    lax.fori_loop(0, K, _zrow, 0)
    for t in range(RPS // K):
        pltpu.sync_copy(rows0, acc_sh.at[pl.ds(sid * RPS + t * K, K)])
    plsc.subcore_barrier()

    o16 = jnp.ones((LANES,), jnp.int32)
    ebase = wid * EPW

    def _weights(idx2):
        bv = b_t[...]
        for v in range(K // LANES):
            s16 = idx2[0, pl.ds(v * LANES, LANES)]
            d16 = idx2[1, pl.ds(v * LANES, LANES)]
            ai = plsc.load_gather(aij_t, (d16 * 2,))
            aj = plsc.load_gather(aij_t, (s16 * 2 + o16,))
            s = ai + aj
            al = jnp.where(s >= 0, s, 0.2 * s)
            w = jnp.exp(al - bv)
            w = jnp.where(s16 == d16, jnp.float32(0.0), w)
            w_t[pl.ds(v * LANES, LANES)] = w

    def _scale_rows(rows):
        def _scale(v, c2):
            w16 = w_t[pl.ds(v * LANES, LANES)]
            for lane in range(LANES):
                wl = _lane_bcast(w16, lane)
                j = v * LANES + lane
                tail = rows[j, pl.ds(D - LANES, LANES)]
                for c in range(C // LANES):
                    rows[j, pl.ds(c * LANES, LANES)] = \
                        rows[j, pl.ds(c * LANES, LANES)] * wl
                rows[j, pl.ds(D - LANES, LANES)] = tail * wl
            return c2

        lax.fori_loop(0, K // LANES, _scale, 0)

    idx2P, rowsP, sgP, ssP, siP = bufs[0]
    pltpu.sync_copy(edges_hbm.at[:, pl.ds(ebase, K)], idx2P)
    pltpu.async_copy(xaug_hbm.at[idx2P.at[0]], rowsP, sgP)

    def _iter(i2, b):
        idx2, rows, sg, ss, si = bufs[b]
        oidx2, orows, osg, oss, osi = bufs[1 - b]
        nbase = ebase + (i2 + 1) * K

        if b == 0:
            @pl.when(i2 > 0)
            def _():
                pltpu.make_async_copy(orows, acc_sh.at[oidx2.at[1]], oss).wait()
        else:
            pltpu.make_async_copy(orows, acc_sh.at[oidx2.at[1]], oss).wait()
        pltpu.async_copy(edges_hbm.at[:, pl.ds(nbase, K)], oidx2, osi)
        _weights(idx2)
        pltpu.make_async_copy(xaug_hbm.at[idx2.at[0]], rows, sg).wait()
        _scale_rows(rows)
        pltpu.make_async_copy(edges_hbm.at[:, pl.ds(nbase, K)], oidx2, osi).wait()
        pltpu.async_copy(xaug_hbm.at[oidx2.at[0]], orows, osg)
        pltpu.async_copy(rows, acc_sh.at[idx2.at[1]], ss, add=True)

    def _outer(g, carry):
        for b in range(2):
            _iter(g * 2 + b, b)
        return carry

    lax.fori_loop(0, (STEPS - 1) // 2, _outer, 0)

    idx2E, rowsE, sgE, ssE, siE = bufs[0]
    pltpu.make_async_copy(rows1, acc_sh.at[idx21.at[1]], ss1).wait()
    _weights(idx2E)
    pltpu.make_async_copy(xaug_hbm.at[idx2E.at[0]], rowsE, sgE).wait()
    _scale_rows(rowsE)
    pltpu.sync_copy(rowsE, acc_sh.at[idx2E.at[1]], add=True)

    plsc.subcore_barrier()
    pltpu.sync_copy(acc_sh.at[pl.ds(sid * RPS, RPS)],
                    acc_hbm.at[cid, pl.ds(sid * RPS, RPS)])


_sc_edges = pl.kernel(
    _sc_body,
    out_type=jax.ShapeDtypeStruct((NC, NP, D), jnp.float32),
    mesh=plsc.VectorSubcoreMesh(core_axis_name="c", subcore_axis_name="s"),
    compiler_params=pltpu.CompilerParams(needs_layout_passes=False, use_tc_tiling_on_sc=False),
    scratch_types=[
        pltpu.VMEM((2 * N,), jnp.float32),
        pltpu.VMEM((2, K), jnp.int32),
        pltpu.VMEM((K, D), jnp.float32),
        pltpu.VMEM((2, K), jnp.int32),
        pltpu.VMEM((K, D), jnp.float32),
        pltpu.VMEM((K,), jnp.float32),
        pltpu.VMEM((LANES,), jnp.float32),
        pltpu.VMEM_SHARED((NP, D), jnp.float32),
        pltpu.SemaphoreType.DMA,
        pltpu.SemaphoreType.DMA,
        pltpu.SemaphoreType.DMA,
        pltpu.SemaphoreType.DMA,
        pltpu.SemaphoreType.DMA,
        pltpu.SemaphoreType.DMA,
    ],
)


def _stage3_body(acc_ref, xaug_ref, aij_ref, bs_ref, bias_ref, gamma_ref,
                 beta_ref, out_ref):
    B = bs_ref[0, 0]
    x = xaug_ref[:, :C]
    s = aij_ref[:, 0:1] + aij_ref[:, 1:2]
    al = jnp.where(s >= 0, s, 0.2 * s)
    exs = jnp.exp(al - B)
    num = acc_ref[0, :N, :C] + acc_ref[1, :N, :C] + exs * x
    den = acc_ref[0, :N, C:C + 1] + acc_ref[1, :N, C:C + 1] + exs
    o = num / jnp.maximum(den, 1e-16) + bias_ref[...][None, :]
    mean = jnp.mean(o, axis=0, keepdims=True)
    var = jnp.mean((o - mean) ** 2, axis=0, keepdims=True)
    o = (o - mean) / jnp.sqrt(var + 1e-5) * gamma_ref[...][None, :] + \
        beta_ref[...][None, :]
    out_ref[...] = jnp.maximum(o, 0.0)


_stage3 = pl.pallas_call(
    _stage3_body,
    out_shape=jax.ShapeDtypeStruct((N, C), jnp.float32),
)


def kernel(batch_mat, topk_edge, embedding, W, att_i, att_j, att_em_i,
           att_em_j, bias, gamma, beta):
    xaug, aij, bmax = _stage1(batch_mat, embedding, W, att_i, att_j,
                              att_em_i, att_em_j)
    ssum = bmax[0, 0] + bmax[0, 1]
    B = jnp.where(ssum >= 0, ssum, 0.2 * ssum)
    acc = _sc_edges(xaug, aij.reshape(2 * N), topk_edge,
                    jnp.broadcast_to(B, (LANES,)))
    return _stage3(acc, xaug, aij, B.reshape(1, 1), bias, gamma, beta)
